# baseline (speedup 1.0000x reference)
import functools

import jax
import jax.numpy as jnp
from jax import lax
from jax.experimental import pallas as pl
from jax.experimental.pallas import tpu as pltpu
from jax.experimental.pallas import tpu_sc as plsc

_DEPTH = 1000
_NROWS = 4096 * 26
_NW = 32
_ROWS_W = _NROWS // _NW
_C = 32
_NCHUNK = _ROWS_W // _C


def _scatter16(buf, row_base, vals_ref, val_off, value):
    for j in range(_C // 16):
        idx16 = vals_ref[pl.ds(val_off + j * 16, 16)]
        rows16 = row_base + j * 16 + lax.broadcasted_iota(jnp.int32, (16,), 0)
        plsc.store_scatter(buf, [rows16 * _DEPTH + idx16],
                           jnp.full((16,), value, jnp.float32))


def _sc_body(x_hbm, out_hbm, vals, buf, sem0, sem1):
    wid = lax.axis_index("s") * 2 + lax.axis_index("c")
    row0 = wid * _ROWS_W

    pltpu.sync_copy(x_hbm.at[pl.ds(row0, _ROWS_W)], vals)

    def _zero_blk(r, _):
        for c0 in range(0, 1024, 16):
            buf[pl.ds(r * 1024 + c0, 16)] = jnp.zeros((16,), jnp.float32)
        return 0

    lax.fori_loop(0, 2 * _C * _DEPTH // 1024, _zero_blk, 0)
    for c0 in range(63488, 64000, 16):
        buf[pl.ds(c0, 16)] = jnp.zeros((16,), jnp.float32)

    def _start(g, b, sem):
        _scatter16(buf, b * _C, vals, g * _C, 1.0)
        return pltpu.async_copy(
            buf.at[pl.ds(b * _C * _DEPTH, _C * _DEPTH)],
            out_hbm.at[pl.ds((row0 + g * _C) * _DEPTH, _C * _DEPTH)],
            sem,
        )

    _start(0, 0, sem0)
    _start(1, 1, sem1)

    def _step(i, _):
        for b, sem in ((0, sem0), (1, sem1)):
            g = 2 * i + b
            pltpu.make_async_copy(
                buf.at[pl.ds(b * _C * _DEPTH, _C * _DEPTH)],
                out_hbm.at[pl.ds((row0 + (g - 2) * _C) * _DEPTH, _C * _DEPTH)],
                sem,
            ).wait()
            _scatter16(buf, b * _C, vals, (g - 2) * _C, 0.0)
            _start(g, b, sem)
        return 0

    lax.fori_loop(1, _NCHUNK // 2, _step, 0)

    for b, sem in ((0, sem0), (1, sem1)):
        g = _NCHUNK - 2 + b
        pltpu.make_async_copy(
            buf.at[pl.ds(b * _C * _DEPTH, _C * _DEPTH)],
            out_hbm.at[pl.ds((row0 + g * _C) * _DEPTH, _C * _DEPTH)],
            sem,
        ).wait()


_sc_one_hot = functools.partial(
    pl.kernel,
    out_type=jax.ShapeDtypeStruct((_NROWS * _DEPTH,), jnp.float32),
    mesh=plsc.VectorSubcoreMesh(core_axis_name="c", subcore_axis_name="s"),
    scratch_types=[
        pltpu.VMEM((_ROWS_W,), jnp.int32),
        pltpu.VMEM((2 * _C * _DEPTH,), jnp.float32),
        pltpu.SemaphoreType.DMA,
        pltpu.SemaphoreType.DMA,
    ],
    compiler_params=pltpu.CompilerParams(needs_layout_passes=False),
)(_sc_body)


def kernel(inputs):
    x = inputs.astype(jnp.int32).reshape(_NROWS)
    out = _sc_one_hot(x)
    return out.reshape(inputs.shape[0], inputs.shape[1], _DEPTH)


# speedup vs baseline: 9.3357x; 9.3357x over previous
import jax
import jax.numpy as jnp
from jax.experimental import pallas as pl

_DEPTH = 1000
_N = 4096
_DBLK = 200


def _body(inp_ref, out_ref):
    j = pl.program_id(1)
    col = inp_ref[...]
    iota = jax.lax.broadcasted_iota(jnp.int32, (1, _DBLK, _N), 1) + j * _DBLK
    out_ref[...] = (iota == col).astype(jnp.float32)


def kernel(inputs):
    xt = inputs.astype(jnp.int32).T.reshape(26, 1, _N)
    out_t = pl.pallas_call(
        _body,
        grid=(26, _DEPTH // _DBLK),
        in_specs=[pl.BlockSpec((1, 1, _N), lambda c, j: (c, 0, 0))],
        out_specs=pl.BlockSpec((1, _DBLK, _N), lambda c, j: (c, j, 0)),
        out_shape=jax.ShapeDtypeStruct((26, _DEPTH, _N), jnp.float32),
    )(xt)
    return jnp.transpose(out_t, (2, 0, 1))
